# Initial kernel scaffold; baseline (speedup 1.0000x reference)
import jax
import jax.numpy as jnp
from jax.experimental import pallas as pl


def kernel(x, token_table, pos_table):
    raise NotImplementedError("write your pallas kernel here")


# speedup vs baseline: 1.8972x; 1.8972x over previous
import functools

import jax
import jax.numpy as jnp
from jax import lax
from jax.experimental import pallas as pl
from jax.experimental.pallas import tpu as pltpu
from jax.experimental.pallas import tpu_sc as plsc

_L = 16


@functools.lru_cache(maxsize=None)
def _make_sc_kernel(n_rows: int, seq_len: int, d: int):
    info = plsc.get_sparse_core_info()
    nc, ns = info.num_cores, info.num_subcores
    nw = nc * ns
    rows_per_w = n_rows // nw
    chunk = 128
    n_chunks = rows_per_w // chunk

    mesh = plsc.VectorSubcoreMesh(core_axis_name="c", subcore_axis_name="s")

    @functools.partial(
        pl.kernel,
        mesh=mesh,
        compiler_params=pltpu.CompilerParams(use_tc_tiling_on_sc=False),
        out_type=jax.ShapeDtypeStruct((n_rows, d), jnp.float32),
        scratch_types=[
            pltpu.VMEM((seq_len, d), jnp.float32),
            pltpu.VMEM((chunk,), jnp.int32),
            pltpu.VMEM((chunk, d), jnp.float32),
            pltpu.SemaphoreType.DMA,
        ],
    )
    def k(x_hbm, tok_hbm, pos_hbm, out_hbm, pos_v, idx_v, rows_v, sem):
        wid = lax.axis_index("s") * nc + lax.axis_index("c")
        base = wid * rows_per_w
        pltpu.sync_copy(pos_hbm, pos_v)

        def chunk_body(ci, carry):
            r0 = base + ci * chunk
            pltpu.sync_copy(x_hbm.at[pl.ds(r0, chunk)], idx_v)
            pltpu.async_copy(tok_hbm.at[idx_v], rows_v, sem).wait()
            p0 = lax.rem(ci * chunk, seq_len)

            def row_body(r, carry2):
                pr = p0 + r
                pr = jnp.where(pr >= seq_len, pr - seq_len, pr)
                for c in range(d // _L):
                    sl = pl.ds(c * _L, _L)
                    rows_v[r, sl] = rows_v[r, sl] + pos_v[pr, sl]
                return carry2

            lax.fori_loop(0, chunk, row_body, 0)
            pltpu.sync_copy(rows_v, out_hbm.at[pl.ds(r0, chunk)])
            return carry

        lax.fori_loop(0, n_chunks, chunk_body, 0)

    return k


def kernel(x, token_table, pos_table):
    b, s = x.shape
    d = token_table.shape[1]
    x_flat = x.reshape(-1).astype(jnp.int32)
    out = _make_sc_kernel(b * s, s, d)(x_flat, token_table, pos_table)
    return out.reshape(b, s, d)


# speedup vs baseline: 2.3954x; 1.2626x over previous
import functools

import jax
import jax.numpy as jnp
from jax import lax
from jax.experimental import pallas as pl
from jax.experimental.pallas import tpu as pltpu
from jax.experimental.pallas import tpu_sc as plsc

_L = 16
_CHUNK = 128
_NBUF = 5


@functools.lru_cache(maxsize=None)
def _make_sc_kernel(n_rows: int, seq_len: int, d: int):
    info = plsc.get_sparse_core_info()
    nc, ns = info.num_cores, info.num_subcores
    nw = nc * ns
    rows_per_w = n_rows // nw
    n_chunks = rows_per_w // _CHUNK

    mesh = plsc.VectorSubcoreMesh(core_axis_name="c", subcore_axis_name="s")

    @functools.partial(
        pl.kernel,
        mesh=mesh,
        compiler_params=pltpu.CompilerParams(use_tc_tiling_on_sc=False),
        out_type=jax.ShapeDtypeStruct((n_rows, d), jnp.float32),
        scratch_types=[
            pltpu.VMEM((seq_len, d), jnp.float32),
            pltpu.VMEM((n_chunks, _CHUNK), jnp.int32),
            [pltpu.VMEM((_CHUNK, d), jnp.float32)] * _NBUF,
            [pltpu.SemaphoreType.DMA] * _NBUF,
            [pltpu.SemaphoreType.DMA] * _NBUF,
        ],
    )
    def k(x_hbm, tok_hbm, pos_hbm, out_hbm, pos_v, idx_v, bufs, gsems, ssems):
        wid = lax.axis_index("s") * nc + lax.axis_index("c")
        base = wid * rows_per_w
        pltpu.sync_copy(pos_hbm, pos_v)
        pltpu.sync_copy(x_hbm.at[wid], idx_v)

        def gather_start(g, slot):
            pltpu.async_copy(tok_hbm.at[idx_v.at[g]], bufs[slot], gsems[slot])

        def gather_wait(g, slot):
            pltpu.make_async_copy(
                tok_hbm.at[idx_v.at[g]], bufs[slot], gsems[slot]
            ).wait()

        def out_slice(g):
            return out_hbm.at[pl.ds(base + g * _CHUNK, _CHUNK)]

        for g in range(2):
            gather_start(g, g)

        def outer(oi, carry):
            for b in range(_NBUF):
                g = oi * _NBUF + b
                pslot = (b + 2) % _NBUF

                @pl.when(g + 2 < n_chunks)
                def _():
                    @pl.when(g - 3 >= 0)
                    def _():
                        pltpu.make_async_copy(
                            bufs[pslot], out_slice(g - 3), ssems[pslot]
                        ).wait()

                    gather_start(g + 2, pslot)

                gather_wait(g, b)

                p0 = lax.rem(g * _CHUNK, seq_len)

                def row_body(r, carry2):
                    pr = p0 + r
                    pr = jnp.where(pr >= seq_len, pr - seq_len, pr)
                    buf = bufs[b]
                    for c in range(d // _L):
                        sl = pl.ds(c * _L, _L)
                        buf[r, sl] = buf[r, sl] + pos_v[pr, sl]
                    return carry2

                lax.fori_loop(0, _CHUNK, row_body, 0)
                pltpu.async_copy(bufs[b], out_slice(g), ssems[b])
            return carry

        lax.fori_loop(0, n_chunks // _NBUF, outer, 0)

        for b in range(_NBUF):
            g = n_chunks - _NBUF + b
            pltpu.make_async_copy(bufs[b], out_slice(g), ssems[b]).wait()

    return k


def kernel(x, token_table, pos_table):
    b, s = x.shape
    d = token_table.shape[1]
    n_rows = b * s
    info = plsc.get_sparse_core_info()
    nw = info.num_cores * info.num_subcores
    n_chunks = n_rows // nw // _CHUNK
    x_idx = x.astype(jnp.int32).reshape(nw, n_chunks, _CHUNK)
    out = _make_sc_kernel(n_rows, s, d)(x_idx, token_table, pos_table)
    return out.reshape(b, s, d)
